# Initial kernel scaffold; baseline (speedup 1.0000x reference)
import jax
import jax.numpy as jnp
from jax.experimental import pallas as pl

N = 10000
E = 320000
D_IN = 128
D_EDGE = 16
H = 4
C = 32
EO = 32
NEG_SLOPE = 0.2


def _bias_add_body(out_ref, nb_ref, eb_ref, o_ref):
    o_ref[...] = out_ref[...] + nb_ref[...] + eb_ref[...]


def kernel(x, edge_attr, edge_index, W_l, W_e, att_l, att_r, att_e, node_bias, edge_bias):
    xl = (x @ W_l).reshape(-1, H, C)
    alpha_l = jnp.sum(xl * att_l, axis=-1)
    alpha_r = jnp.sum(xl * att_r, axis=-1)
    e = (edge_attr @ W_e).reshape(-1, H, EO)
    alpha_e = jnp.sum(e * att_e, axis=-1)

    src = edge_index[0]
    dst = edge_index[1]

    alpha = alpha_l[src] + alpha_r[dst] + alpha_e
    alpha = jnp.where(alpha >= 0, alpha, NEG_SLOPE * alpha)

    amax = jax.ops.segment_max(alpha, dst, num_segments=N)
    amax = jnp.where(jnp.isfinite(amax), amax, 0.0)
    ex = jnp.exp(alpha - amax[dst])
    denom = jax.ops.segment_sum(ex, dst, num_segments=N)
    alpha_n = ex / (denom[dst] + 1e-16)

    msg = xl[src] * alpha_n[:, :, None]
    out = jax.ops.segment_sum(msg, dst, num_segments=N)
    out = out.reshape(-1, H * C)

    out = pl.pallas_call(
        _bias_add_body,
        out_shape=jax.ShapeDtypeStruct((N, H * C), jnp.float32),
    )(out, jnp.broadcast_to(node_bias, (N, H * C)), jnp.broadcast_to(edge_bias, (N, H * C)))
    return (out, out)


# speedup vs baseline: 24.6203x; 24.6203x over previous
import functools

import jax
import jax.numpy as jnp
from jax import lax
from jax.experimental import pallas as pl
from jax.experimental.pallas import tpu as pltpu
from jax.experimental.pallas import tpu_sc as plsc

N = 10000
E = 320000
D_IN = 128
H = 4
C = 32
NEG_SLOPE = 0.2

NP = 10240
DT = 2 * NP
BN = 1000
BE = 2000
K1 = 400
K2 = 80
EPT = E // 16
RPT = NP // 16


def _node_body(x_ref, w_ref, xl_ref):
    xl_ref[...] = jnp.dot(x_ref[...], w_ref[0],
                          preferred_element_type=jnp.float32)


def _alr_body(xl0_ref, xl1_ref, p_ref, lr_ref):
    lr0 = jnp.dot(xl0_ref[...], p_ref[0], preferred_element_type=jnp.float32)
    lr1 = jnp.dot(xl1_ref[...], p_ref[1], preferred_element_type=jnp.float32)
    lr_ref[...] = jnp.concatenate([lr0, lr1], axis=1)


def _edge_body(ea_ref, v_ref, ae_ref):
    ae_ref[...] = jnp.dot(ea_ref[...], v_ref[...],
                          preferred_element_type=jnp.float32)


def _sc_body(alT_h, arT_h, ae_h, src_h, dst_h, xl_h, mvec_h, bias_h, out_h,
             al_tab, ar_tab, den_flat, src1, dst1, ae1,
             srcoff, dstidx, rows, mvec_v, bias_v, biasblk,
             mbuf, macc, dstage_h, den_final, acc):
    p = lax.axis_index("c")
    s = lax.axis_index("s")
    fzero = jnp.zeros((16,), jnp.float32)
    izero = jnp.zeros((16,), jnp.int32)
    iota16 = lax.iota(jnp.int32, 16)

    pltpu.sync_copy(alT_h.at[pl.ds(p * DT, DT)], al_tab)
    pltpu.sync_copy(arT_h.at[pl.ds(p * DT, DT)], ar_tab)
    pltpu.sync_copy(mvec_h, mvec_v)
    pltpu.sync_copy(bias_h.at[p], bias_v)
    mvec = mvec_v[...]

    def _zden(i, carry):
        den_flat[pl.ds(i * 16, 16)] = fzero
        return carry
    lax.fori_loop(0, DT // 16, _zden, 0)

    bias_regs = [bias_v[0, pl.ds(q * 16, 16)] for q in range(4)]

    def _binit(i, carry):
        for q in range(4):
            biasblk[i, pl.ds(q * 16, 16)] = bias_regs[q]
        return carry
    lax.fori_loop(0, 16, _binit, 0)
    for k in range(RPT // 16):
        pltpu.sync_copy(biasblk, acc.at[pl.ds(s * RPT + k * 16, 16), :])

    def _den_chunk(c, carry):
        base = s * EPT + c * K1
        pltpu.sync_copy(src_h.at[pl.ds(base, K1)], src1)
        pltpu.sync_copy(dst_h.at[pl.ds(base, K1)], dst1)
        pltpu.sync_copy(ae_h.at[pl.ds(base * 8, K1 * 8)], ae1)

        def _den_grp(g, carry2):
            sv = src1[pl.ds(g * 16, 16)]
            dv = dst1[pl.ds(g * 16, 16)]
            rows_i = iota16 + g * 16
            for h in (0, 1):
                al = plsc.load_gather(al_tab, [sv + h * NP])
                ar = plsc.load_gather(ar_tab, [dv + h * NP])
                ae = plsc.load_gather(ae1, [rows_i * 8 + (2 * p + h)])
                sm = al + ar + ae
                sm = jnp.where(sm >= 0, sm, NEG_SLOPE * sm)
                ex = jnp.exp(sm - mvec)
                plsc.addupdate_scatter(den_flat, [dv + h * NP], ex)
            return carry2
        return lax.fori_loop(0, K1 // 16, _den_grp, carry)
    lax.fori_loop(0, EPT // K1, _den_chunk, 0)

    pltpu.sync_copy(den_flat, dstage_h.at[p, s])
    plsc.subcore_barrier()
    sub = s * (DT // 16)
    pltpu.sync_copy(dstage_h.at[p, 0, pl.ds(sub, DT // 16)], macc)
    for t in range(1, 16):
        pltpu.sync_copy(dstage_h.at[p, t, pl.ds(sub, DT // 16)], mbuf)

        def _madd(i, carry, _t=t):
            macc[pl.ds(i * 16, 16)] = (macc[pl.ds(i * 16, 16)]
                                       + mbuf[pl.ds(i * 16, 16)])
            return carry
        lax.fori_loop(0, DT // 256, _madd, 0)

    def _meps(i, carry):
        macc[pl.ds(i * 16, 16)] = macc[pl.ds(i * 16, 16)] + 1e-16
        return carry
    lax.fori_loop(0, DT // 256, _meps, 0)
    pltpu.sync_copy(macc, den_final.at[pl.ds(sub, DT // 16)])
    plsc.subcore_barrier()
    pltpu.sync_copy(den_final, den_flat)

    def _agg_chunk(c, carry):
        base = s * EPT + c * K1
        pltpu.sync_copy(src_h.at[pl.ds(base, K1)], src1)
        pltpu.sync_copy(dst_h.at[pl.ds(base, K1)], dst1)
        pltpu.sync_copy(ae_h.at[pl.ds(base * 8, K1 * 8)], ae1)

        def _agg_sub(j, carry2):
            b = jnp.bitwise_and(j, 1)
            off = j * K2
            for g in range(K2 // 16):
                sv = src1[pl.ds(off + g * 16, 16)]
                dv = dst1[pl.ds(off + g * 16, 16)]
                srcoff[b, 0, pl.ds(g * 16, 16)] = sv + p * N
                dstidx[b, 0, pl.ds(g * 16, 16)] = dv
            pltpu.sync_copy(xl_h.at[srcoff.at[b, 0]], rows.at[b])
            for g in range(K2 // 16):
                sv = src1[pl.ds(off + g * 16, 16)]
                dv = dst1[pl.ds(off + g * 16, 16)]
                rows_i = iota16 + (off + g * 16)
                an = []
                for h in (0, 1):
                    al = plsc.load_gather(al_tab, [sv + h * NP])
                    ar = plsc.load_gather(ar_tab, [dv + h * NP])
                    ae = plsc.load_gather(ae1, [rows_i * 8 + (2 * p + h)])
                    sm = al + ar + ae
                    sm = jnp.where(sm >= 0, sm, NEG_SLOPE * sm)
                    ex = jnp.exp(sm - mvec)
                    den = plsc.load_gather(den_flat, [dv + h * NP])
                    an.append(ex / den)
                for l in range(16):
                    e = g * 16 + l
                    for q in range(4):
                        sc = an[0][l] if q < 2 else an[1][l]
                        rows[b, e, pl.ds(q * 16, 16)] = (
                            rows[b, e, pl.ds(q * 16, 16)] * sc)
            pltpu.sync_copy(rows.at[b], acc.at[dstidx.at[b, 0]], add=True)
            return carry2
        return lax.fori_loop(0, K1 // K2, _agg_sub, carry)
    lax.fori_loop(0, EPT // K1, _agg_chunk, 0)

    plsc.subcore_barrier()
    pltpu.sync_copy(acc.at[pl.ds(s * RPT, RPT), :],
                    out_h.at[p, pl.ds(s * RPT, RPT), :])


_sc_call = functools.partial(
    pl.kernel,
    out_type=jax.ShapeDtypeStruct((2, NP, 64), jnp.float32),
    mesh=plsc.VectorSubcoreMesh(core_axis_name="c", subcore_axis_name="s"),
    compiler_params=pltpu.CompilerParams(needs_layout_passes=False,
                                         use_tc_tiling_on_sc=False),
    scratch_types=[
        pltpu.VMEM((DT,), jnp.float32),
        pltpu.VMEM((DT,), jnp.float32),
        pltpu.VMEM((DT,), jnp.float32),
        pltpu.VMEM((K1,), jnp.int32),
        pltpu.VMEM((K1,), jnp.int32),
        pltpu.VMEM((K1 * 8,), jnp.float32),
        pltpu.VMEM((2, 1, K2), jnp.int32),
        pltpu.VMEM((2, 1, K2), jnp.int32),
        pltpu.VMEM((2, K2, 64), jnp.float32),
        pltpu.VMEM((16,), jnp.float32),
        pltpu.VMEM((1, 64), jnp.float32),
        pltpu.VMEM((16, 64), jnp.float32),
        pltpu.VMEM((DT // 16,), jnp.float32),
        pltpu.VMEM((DT // 16,), jnp.float32),
        pltpu.HBM((2, 16, DT), jnp.float32),
        pltpu.VMEM_SHARED((DT,), jnp.float32),
        pltpu.VMEM_SHARED((NP, 64), jnp.float32),
    ],
)(_sc_body)


def kernel(x, edge_attr, edge_index, W_l, W_e, att_l, att_r, att_e, node_bias, edge_bias):
    f32 = jnp.float32
    V = jnp.einsum("dho,ho->dh", W_e.reshape(16, H, 32), att_e[0])
    V8 = jnp.concatenate([V, jnp.broadcast_to(V[:, :1], (16, 4))], axis=1)

    W_pairs = W_l.reshape(D_IN, 2, 64).transpose(1, 0, 2)
    eyeh = jnp.repeat(jnp.eye(2, dtype=f32), C, axis=0)
    P = jnp.zeros((2, 64, 8), f32)
    for pair in range(2):
        att_lp = att_l[0, 2 * pair:2 * pair + 2].reshape(2, C)
        att_rp = att_r[0, 2 * pair:2 * pair + 2].reshape(2, C)
        colsl = eyeh * att_lp.reshape(64)[:, None]
        colsr = eyeh * att_rp.reshape(64)[:, None]
        blk = jnp.concatenate(
            [colsl, colsr, jnp.broadcast_to(colsl[:, :1], (64, 4))], axis=1)
        P = P.at[pair].set(blk)

    nb = N // BN
    xl_tab = pl.pallas_call(
        _node_body,
        grid=(2, nb),
        in_specs=[
            pl.BlockSpec((BN, D_IN), lambda p, i: (i, 0)),
            pl.BlockSpec((1, D_IN, 64), lambda p, i: (p, 0, 0)),
        ],
        out_specs=pl.BlockSpec((BN, 64), lambda p, i: (p * (N // BN) + i, 0)),
        out_shape=jax.ShapeDtypeStruct((2 * N, 64), f32),
    )(x, W_pairs)

    alr = pl.pallas_call(
        _alr_body,
        grid=(nb,),
        in_specs=[
            pl.BlockSpec((BN, 64), lambda i: (i, 0)),
            pl.BlockSpec((BN, 64), lambda i: (nb + i, 0)),
            pl.BlockSpec((2, 64, 8), lambda i: (0, 0, 0)),
        ],
        out_specs=pl.BlockSpec((BN, 16), lambda i: (i, 0)),
        out_shape=jax.ShapeDtypeStruct((N, 16), f32),
    )(xl_tab, xl_tab, P)

    ae8 = pl.pallas_call(
        _edge_body,
        grid=(E // BE,),
        in_specs=[
            pl.BlockSpec((BE, 16), lambda i: (i, 0)),
            pl.BlockSpec((16, 8), lambda i: (0, 0)),
        ],
        out_specs=pl.BlockSpec((BE, 8), lambda i: (i, 0)),
        out_shape=jax.ShapeDtypeStruct((E, 8), f32),
    )(edge_attr, V8)

    m0 = 2.0 * jnp.max(alr) + jnp.max(ae8)
    M = jnp.maximum(m0, NEG_SLOPE * m0)
    mvec = jnp.full((16,), M, f32)

    alT = jnp.zeros((4, NP), f32).at[:, :N].set(alr[:, (0, 1, 8, 9)].T).reshape(4 * NP)
    arT = jnp.zeros((4, NP), f32).at[:, :N].set(alr[:, (2, 3, 10, 11)].T).reshape(4 * NP)

    bias2 = (node_bias + edge_bias).reshape(2, 1, 64)
    src = edge_index[0]
    dst = edge_index[1]

    out3 = _sc_call(alT, arT, ae8.reshape(E * 8), src, dst, xl_tab, mvec, bias2)
    out = out3[:, :N].transpose(1, 0, 2).reshape(N, H * C)
    return (out, out)


# speedup vs baseline: 28.1026x; 1.1414x over previous
import functools

import jax
import jax.numpy as jnp
from jax import lax
from jax.experimental import pallas as pl
from jax.experimental.pallas import tpu as pltpu
from jax.experimental.pallas import tpu_sc as plsc

N = 10000
E = 320000
D_IN = 128
H = 4
C = 32
NEG_SLOPE = 0.2

NP = 10240
DT = 2 * NP
BN = 1000
BE = 2000
K1 = 400
K2 = 80
EPT = E // 16
RPT = NP // 16


def _node_body(x_ref, w_ref, xl_ref):
    xl_ref[...] = jnp.dot(x_ref[...], w_ref[0],
                          preferred_element_type=jnp.float32)


def _alr_body(xl0_ref, xl1_ref, p_ref, lr_ref):
    lr0 = jnp.dot(xl0_ref[...], p_ref[0], preferred_element_type=jnp.float32)
    lr1 = jnp.dot(xl1_ref[...], p_ref[1], preferred_element_type=jnp.float32)
    lr_ref[...] = jnp.concatenate([lr0, lr1], axis=1)


def _edge_body(ea_ref, v_ref, ae_ref):
    ae_ref[...] = jnp.dot(ea_ref[...], v_ref[...],
                          preferred_element_type=jnp.float32)


def _sc_body(alT_h, arT_h, ae_h, src_h, dst_h, xl_h, mvec_h, bias_h, out_h,
             al_tab, ar_tab, den_flat, src1, dst1, ae1,
             srcoff, dstidx, rows, mvec_v, bias_v, biasblk,
             mbuf, exbuf, idxbuf, dsem, gsem, den_final, acc):
    p = lax.axis_index("c")
    s = lax.axis_index("s")
    fzero = jnp.zeros((16,), jnp.float32)
    izero = jnp.zeros((16,), jnp.int32)
    iota16 = lax.iota(jnp.int32, 16)

    pltpu.sync_copy(alT_h.at[pl.ds(p * DT, DT)], al_tab)
    pltpu.sync_copy(arT_h.at[pl.ds(p * DT, DT)], ar_tab)
    pltpu.sync_copy(mvec_h, mvec_v)
    pltpu.sync_copy(bias_h.at[p], bias_v)
    mvec = mvec_v[...]

    bias_regs = [bias_v[0, pl.ds(q * 16, 16)] for q in range(4)]

    def _binit(i, carry):
        for q in range(4):
            biasblk[i, pl.ds(q * 16, 16)] = bias_regs[q]
        return carry
    lax.fori_loop(0, 16, _binit, 0)
    for k in range(RPT // 16):
        pltpu.sync_copy(biasblk, acc.at[pl.ds(s * RPT + k * 16, 16), :])

    sub = s * (DT // 16)

    def _zmb(i, carry):
        mbuf[pl.ds(i * 16, 16)] = fzero
        return carry
    lax.fori_loop(0, DT // 256, _zmb, 0)
    pltpu.sync_copy(mbuf, den_final.at[pl.ds(sub, DT // 16)])
    plsc.subcore_barrier()

    def _den_chunk(c, carry):
        base = s * EPT + c * K1
        pltpu.sync_copy(src_h.at[pl.ds(base, K1)], src1)
        pltpu.sync_copy(dst_h.at[pl.ds(base, K1)], dst1)
        pltpu.sync_copy(ae_h.at[pl.ds(base * 8, K1 * 8)], ae1)

        def _den_grp(g, carry2):
            sv = src1[pl.ds(g * 16, 16)]
            dv = dst1[pl.ds(g * 16, 16)]
            rows_i = iota16 + g * 16
            r_lo = g // 5
            c_lo = (g % 5) * 16
            for h in (0, 1):
                al = plsc.load_gather(al_tab, [sv + h * NP])
                ar = plsc.load_gather(ar_tab, [dv + h * NP])
                ae = plsc.load_gather(ae1, [rows_i * 8 + (2 * p + h)])
                sm = al + ar + ae
                sm = jnp.where(sm >= 0, sm, NEG_SLOPE * sm)
                ex = jnp.exp(sm - mvec)
                exbuf[h * 5 + r_lo, 0, pl.ds(c_lo, 16)] = ex
                idxbuf[h * 5 + r_lo, 0, pl.ds(c_lo, 16)] = dv + h * NP
            return carry2
        lax.fori_loop(0, K1 // 16, _den_grp, carry)
        handles = [
            pltpu.async_copy(exbuf.at[k, 0], den_final.at[idxbuf.at[k, 0]],
                             dsem, add=True)
            for k in range(2 * K1 // 80)
        ]
        for hd in handles:
            hd.wait()
        return carry
    lax.fori_loop(0, EPT // K1, _den_chunk, 0)

    plsc.subcore_barrier()
    pltpu.sync_copy(den_final, den_flat)

    def _deps(i, carry):
        den_flat[pl.ds(i * 16, 16)] = den_flat[pl.ds(i * 16, 16)] + 1e-16
        return carry
    lax.fori_loop(0, DT // 16, _deps, 0)

    def _build_idx(bb, off):
        for g in range(K2 // 16):
            sv = src1[pl.ds(off + g * 16, 16)]
            dv = dst1[pl.ds(off + g * 16, 16)]
            srcoff[bb, 0, pl.ds(g * 16, 16)] = sv + p * N
            dstidx[bb, 0, pl.ds(g * 16, 16)] = dv

    def _agg_chunk(c, carry):
        base = s * EPT + c * K1
        pltpu.sync_copy(src_h.at[pl.ds(base, K1)], src1)
        pltpu.sync_copy(dst_h.at[pl.ds(base, K1)], dst1)
        pltpu.sync_copy(ae_h.at[pl.ds(base * 8, K1 * 8)], ae1)

        def _agg_sub(j, carry2):
            b = jnp.bitwise_and(j, 1)
            bn = 1 - b
            off = j * K2

            @pl.when(j == 0)
            def _prime():
                _build_idx(b, off)
                pltpu.sync_copy(xl_h.at[srcoff.at[b, 0]], rows.at[b])

            @pl.when(j < (K1 // K2) - 1)
            def _prefetch():
                _build_idx(bn, off + K2)
                pltpu.async_copy(xl_h.at[srcoff.at[bn, 0]], rows.at[bn], gsem)

            @pl.when(j > 0)
            def _wait_cur():
                pltpu.make_async_copy(xl_h.at[srcoff.at[b, 0]],
                                      rows.at[b], gsem).wait()

            for g in range(K2 // 16):
                sv = src1[pl.ds(off + g * 16, 16)]
                dv = dst1[pl.ds(off + g * 16, 16)]
                rows_i = iota16 + (off + g * 16)
                an = []
                for h in (0, 1):
                    al = plsc.load_gather(al_tab, [sv + h * NP])
                    ar = plsc.load_gather(ar_tab, [dv + h * NP])
                    ae = plsc.load_gather(ae1, [rows_i * 8 + (2 * p + h)])
                    sm = al + ar + ae
                    sm = jnp.where(sm >= 0, sm, NEG_SLOPE * sm)
                    ex = jnp.exp(sm - mvec)
                    den = plsc.load_gather(den_flat, [dv + h * NP])
                    an.append(ex / den)
                for l in range(16):
                    e = g * 16 + l
                    for q in range(4):
                        sc = an[0][l] if q < 2 else an[1][l]
                        rows[b, e, pl.ds(q * 16, 16)] = (
                            rows[b, e, pl.ds(q * 16, 16)] * sc)
            pltpu.sync_copy(rows.at[b], acc.at[dstidx.at[b, 0]], add=True)
            return carry2
        return lax.fori_loop(0, K1 // K2, _agg_sub, carry)
    lax.fori_loop(0, EPT // K1, _agg_chunk, 0)

    plsc.subcore_barrier()
    pltpu.sync_copy(acc.at[pl.ds(s * RPT, RPT), :],
                    out_h.at[p, pl.ds(s * RPT, RPT), :])


_sc_call = functools.partial(
    pl.kernel,
    out_type=jax.ShapeDtypeStruct((2, NP, 64), jnp.float32),
    mesh=plsc.VectorSubcoreMesh(core_axis_name="c", subcore_axis_name="s"),
    compiler_params=pltpu.CompilerParams(needs_layout_passes=False,
                                         use_tc_tiling_on_sc=False),
    scratch_types=[
        pltpu.VMEM((DT,), jnp.float32),
        pltpu.VMEM((DT,), jnp.float32),
        pltpu.VMEM((DT,), jnp.float32),
        pltpu.VMEM((K1,), jnp.int32),
        pltpu.VMEM((K1,), jnp.int32),
        pltpu.VMEM((K1 * 8,), jnp.float32),
        pltpu.VMEM((2, 1, K2), jnp.int32),
        pltpu.VMEM((2, 1, K2), jnp.int32),
        pltpu.VMEM((2, K2, 64), jnp.float32),
        pltpu.VMEM((16,), jnp.float32),
        pltpu.VMEM((1, 64), jnp.float32),
        pltpu.VMEM((16, 64), jnp.float32),
        pltpu.VMEM((DT // 16,), jnp.float32),
        pltpu.VMEM((2 * K1 // 80, 1, 80), jnp.float32),
        pltpu.VMEM((2 * K1 // 80, 1, 80), jnp.int32),
        pltpu.SemaphoreType.DMA,
        pltpu.SemaphoreType.DMA,
        pltpu.VMEM_SHARED((DT,), jnp.float32),
        pltpu.VMEM_SHARED((NP, 64), jnp.float32),
    ],
)(_sc_body)


def kernel(x, edge_attr, edge_index, W_l, W_e, att_l, att_r, att_e, node_bias, edge_bias):
    f32 = jnp.float32
    V = jnp.einsum("dho,ho->dh", W_e.reshape(16, H, 32), att_e[0])
    V8 = jnp.concatenate([V, jnp.broadcast_to(V[:, :1], (16, 4))], axis=1)

    W_pairs = W_l.reshape(D_IN, 2, 64).transpose(1, 0, 2)
    eyeh = jnp.repeat(jnp.eye(2, dtype=f32), C, axis=0)
    P = jnp.zeros((2, 64, 8), f32)
    for pair in range(2):
        att_lp = att_l[0, 2 * pair:2 * pair + 2].reshape(2, C)
        att_rp = att_r[0, 2 * pair:2 * pair + 2].reshape(2, C)
        colsl = eyeh * att_lp.reshape(64)[:, None]
        colsr = eyeh * att_rp.reshape(64)[:, None]
        blk = jnp.concatenate(
            [colsl, colsr, jnp.broadcast_to(colsl[:, :1], (64, 4))], axis=1)
        P = P.at[pair].set(blk)

    nb = N // BN
    xl_tab = pl.pallas_call(
        _node_body,
        grid=(2, nb),
        in_specs=[
            pl.BlockSpec((BN, D_IN), lambda p, i: (i, 0)),
            pl.BlockSpec((1, D_IN, 64), lambda p, i: (p, 0, 0)),
        ],
        out_specs=pl.BlockSpec((BN, 64), lambda p, i: (p * (N // BN) + i, 0)),
        out_shape=jax.ShapeDtypeStruct((2 * N, 64), f32),
    )(x, W_pairs)

    alr = pl.pallas_call(
        _alr_body,
        grid=(nb,),
        in_specs=[
            pl.BlockSpec((BN, 64), lambda i: (i, 0)),
            pl.BlockSpec((BN, 64), lambda i: (nb + i, 0)),
            pl.BlockSpec((2, 64, 8), lambda i: (0, 0, 0)),
        ],
        out_specs=pl.BlockSpec((BN, 16), lambda i: (i, 0)),
        out_shape=jax.ShapeDtypeStruct((N, 16), f32),
    )(xl_tab, xl_tab, P)

    ae8 = pl.pallas_call(
        _edge_body,
        grid=(E // BE,),
        in_specs=[
            pl.BlockSpec((BE, 16), lambda i: (i, 0)),
            pl.BlockSpec((16, 8), lambda i: (0, 0)),
        ],
        out_specs=pl.BlockSpec((BE, 8), lambda i: (i, 0)),
        out_shape=jax.ShapeDtypeStruct((E, 8), f32),
    )(edge_attr, V8)

    m0 = 2.0 * jnp.max(alr) + jnp.max(ae8)
    M = jnp.maximum(m0, NEG_SLOPE * m0)
    mvec = jnp.full((16,), M, f32)

    alT = jnp.zeros((4, NP), f32).at[:, :N].set(alr[:, (0, 1, 8, 9)].T).reshape(4 * NP)
    arT = jnp.zeros((4, NP), f32).at[:, :N].set(alr[:, (2, 3, 10, 11)].T).reshape(4 * NP)

    bias2 = (node_bias + edge_bias).reshape(2, 1, 64)
    src = edge_index[0]
    dst = edge_index[1]

    out3 = _sc_call(alT, arT, ae8.reshape(E * 8), src, dst, xl_tab, mvec, bias2)
    out = out3[:, :N].transpose(1, 0, 2).reshape(N, H * C)
    return (out, out)
